# baseline (speedup 1.0000x reference)
import jax
import jax.numpy as jnp
from jax.experimental import pallas as pl
from jax.experimental.pallas import tpu as pltpu

_MEDNUM = 2048
_FDIM = 64
_CH = 128
_NCHUNK = _MEDNUM // _CH
_NBUF = 4


def _ddi_gcn_kernel(adj_hbm, w_ref, inter_ref, out_ref, buf_ref, sems):
    w = w_ref[:]
    t = inter_ref[0, 0]

    def copy(i, slot):
        return pltpu.make_async_copy(
            adj_hbm.at[:, pl.ds(i * _CH, _CH), :],
            buf_ref.at[slot],
            sems.at[slot],
        )

    for s in range(_NBUF):
        copy(s, s).start()

    for i in range(_NCHUNK):
        slot = i % _NBUF
        copy(i, slot).wait()
        a1 = buf_ref[slot, 0, :, :_MEDNUM] + buf_ref[slot, 0, :, _MEDNUM:]
        a2 = buf_ref[slot, 1, :, :_MEDNUM] + buf_ref[slot, 1, :, _MEDNUM:]
        y1 = jnp.maximum(jnp.dot(a1, w, preferred_element_type=jnp.float32), 0.0)
        y2 = jnp.maximum(jnp.dot(a2, w, preferred_element_type=jnp.float32), 0.0)
        out_ref[pl.ds(i * _CH, _CH), :] = (2.0 * t) * y1 + (2.0 - 2.0 * t) * y2
        if i + _NBUF < _NCHUNK:
            copy(i + _NBUF, slot).start()


@jax.jit
def kernel(adj1, mEmbed, inter):
    adj3 = adj1.reshape(2, _MEDNUM, 2 * _MEDNUM)
    return pl.pallas_call(
        _ddi_gcn_kernel,
        in_specs=[
            pl.BlockSpec(memory_space=pltpu.HBM),
            pl.BlockSpec(memory_space=pltpu.VMEM),
            pl.BlockSpec(memory_space=pltpu.VMEM),
        ],
        out_specs=pl.BlockSpec(memory_space=pltpu.VMEM),
        out_shape=jax.ShapeDtypeStruct((_MEDNUM, _FDIM), jnp.float32),
        scratch_shapes=[
            pltpu.VMEM((_NBUF, 2, _CH, 2 * _MEDNUM), jnp.float32),
            pltpu.SemaphoreType.DMA((_NBUF,)),
        ],
    )(adj3, mEmbed, inter.reshape(1, 1))
